# Initial kernel scaffold; baseline (speedup 1.0000x reference)
import functools

import jax
import jax.numpy as jnp
from jax import lax
from jax.experimental import pallas as pl
from jax.experimental.pallas import tpu as pltpu
from jax.experimental.pallas import tpu_sc as plsc

N_NODES = 100000
NUM_SEGMENTS = 1024
HIDDEN = 128
LANES = 16
HV = HIDDEN // LANES
NUM_WORKERS = 32
SEG_PER_W = NUM_SEGMENTS // NUM_WORKERS
CHUNK = 192


def _splat(val, dtype=jnp.int32):
    return jnp.full((LANES,), val, dtype)


def _extract_i32(ref, idx_scalar):
    g = plsc.load_gather(ref, [_splat(idx_scalar)])
    return jnp.max(g)


@functools.partial(
    pl.kernel,
    mesh=plsc.VectorSubcoreMesh(core_axis_name="c", subcore_axis_name="s"),
    out_type=jax.ShapeDtypeStruct((NUM_SEGMENTS * HIDDEN,), jnp.float32),
    scratch_types=[
        pltpu.VMEM((CHUNK * HIDDEN,), jnp.float32),
        pltpu.VMEM((CHUNK,), jnp.int32),
        pltpu.VMEM((48,), jnp.int32),
        pltpu.VMEM((16,), jnp.float32),
        pltpu.VMEM(((SEG_PER_W + 1) * HIDDEN,), jnp.float32),
        pltpu.VMEM(((SEG_PER_W + 1) * HIDDEN,), jnp.float32),
        pltpu.VMEM((48,), jnp.float32),
        pltpu.VMEM((SEG_PER_W * HIDDEN,), jnp.float32),
    ],
)
def _readout_kernel(x_hbm, b_hbm, bounds_hbm, w_hbm, out_hbm,
                    xbuf, bbuf, bv, wv, asum, amax, cnt, obuf):
    wid = lax.axis_index("c") * 16 + lax.axis_index("s")
    iota = jnp.arange(LANES, dtype=jnp.int32)
    zeros = jnp.zeros((LANES,), jnp.float32)
    neginf = jnp.full((LANES,), -jnp.inf, jnp.float32)
    ones = jnp.ones((LANES,), jnp.float32)
    lane0 = iota == 0

    pltpu.sync_copy(bounds_hbm, bv)
    pltpu.sync_copy(w_hbm, wv)

    def init_body(i, _):
        asum[pl.ds(i * LANES, LANES)] = zeros
        amax[pl.ds(i * LANES, LANES)] = neginf
        return 0

    lax.fori_loop(0, (SEG_PER_W + 1) * HV, init_body, 0)
    cnt[pl.ds(0, LANES)] = zeros
    cnt[pl.ds(16, LANES)] = zeros
    cnt[pl.ds(32, LANES)] = zeros

    start = _extract_i32(bv, wid)
    end = _extract_i32(bv, wid + 1)
    astart = jnp.bitwise_and(start, -8)
    nchunks = (end - astart + (CHUNK - 1)) // CHUNK
    seg_base = wid * SEG_PER_W

    def chunk_body(k, _):
        base0 = astart + k * CHUNK
        base = pl.multiple_of(jnp.minimum(base0, N_NODES - CHUNK), 8)
        pltpu.sync_copy(x_hbm.at[pl.ds(base * HIDDEN, CHUNK * HIDDEN)], xbuf)
        pltpu.sync_copy(b_hbm.at[pl.ds(base, CHUNK)], bbuf)
        lo = jnp.maximum(start, base0)

        def row_body(r, _):
            gr = base + r
            valid = jnp.logical_and(gr >= lo, gr < end)
            segv = plsc.load_gather(bbuf, [_splat(r)]) - seg_base
            lv = jnp.where(_splat(valid, jnp.bool_), segv, _splat(SEG_PER_W))
            plsc.addupdate_scatter(cnt, [lv], ones, mask=lane0)
            rowb = lv * HIDDEN
            xoff = r * HIDDEN
            for h in range(HV):
                xv = xbuf[pl.ds(xoff + h * LANES, LANES)]
                idxv = rowb + (h * LANES + iota)
                plsc.addupdate_scatter(asum, [idxv], xv)
                mv = plsc.load_gather(amax, [idxv])
                plsc.store_scatter(amax, [idxv], jnp.maximum(mv, xv))
            return 0

        lax.fori_loop(0, CHUNK, row_body, 0)
        return 0

    lax.fori_loop(0, nchunks, chunk_body, 0)

    w0 = plsc.load_gather(wv, [_splat(0)])
    w1 = plsc.load_gather(wv, [_splat(1)])
    w2 = plsc.load_gather(wv, [_splat(2)])

    def out_body(s, _):
        cv = plsc.load_gather(cnt, [_splat(s)])
        has = cv > 0.0
        inv = 1.0 / jnp.maximum(cv, 1.0)
        for h in range(HV):
            off = s * HIDDEN + h * LANES
            sv = asum[pl.ds(off, LANES)]
            mx = jnp.where(has, amax[pl.ds(off, LANES)], zeros)
            obuf[pl.ds(off, LANES)] = (w0 * inv) * sv + w1 * mx + w2 * sv
        return 0

    lax.fori_loop(0, SEG_PER_W, out_body, 0)
    pltpu.sync_copy(obuf, out_hbm.at[pl.ds(seg_base * HIDDEN, SEG_PER_W * HIDDEN)])


def kernel(x, batch, mask, weights):
    del mask
    bounds = jnp.searchsorted(
        batch, jnp.arange(33, dtype=jnp.int32) * SEG_PER_W, side="left"
    ).astype(jnp.int32)
    bounds = jnp.concatenate([bounds, jnp.full((15,), N_NODES, jnp.int32)])
    wpad = jnp.zeros((16,), jnp.float32).at[:3].set(weights)
    out = _readout_kernel(x.reshape(-1), batch, bounds, wpad)
    return out.reshape(NUM_SEGMENTS, HIDDEN)


# speedup vs baseline: 4.1971x; 4.1971x over previous
import functools

import jax
import jax.numpy as jnp
from jax import lax
from jax.experimental import pallas as pl
from jax.experimental.pallas import tpu as pltpu
from jax.experimental.pallas import tpu_sc as plsc

N_NODES = 100000
NUM_SEGMENTS = 1024
HIDDEN = 128
LANES = 16
HV = HIDDEN // LANES
SEG_PER_W = 32
CHUNK = 384
BPAD = 1152


def _splat(val, dtype=jnp.int32):
    return jnp.full((LANES,), val, dtype)


def _extract_i32(ref, idx_scalar):
    g = plsc.load_gather(ref, [_splat(idx_scalar)])
    return jnp.max(g)


@functools.partial(
    pl.kernel,
    mesh=plsc.VectorSubcoreMesh(core_axis_name="c", subcore_axis_name="s"),
    out_type=jax.ShapeDtypeStruct((NUM_SEGMENTS * HIDDEN,), jnp.float32),
    compiler_params=pltpu.CompilerParams(needs_layout_passes=False),
    scratch_types=[
        pltpu.VMEM((CHUNK * HIDDEN,), jnp.float32),
        pltpu.VMEM((BPAD,), jnp.int32),
        pltpu.VMEM((128,), jnp.float32),
        pltpu.VMEM((SEG_PER_W * HIDDEN,), jnp.float32),
        pltpu.VMEM((SEG_PER_W * HIDDEN,), jnp.float32),
        pltpu.VMEM((SEG_PER_W * HIDDEN,), jnp.float32),
        pltpu.SMEM((40,), jnp.int32),
    ],
)
def _readout_kernel(x_hbm, bounds_hbm, w_hbm, out_hbm,
                    xbuf, bv, wv, asum, amax, obuf, bsm):
    wid = lax.axis_index("c") * 16 + lax.axis_index("s")
    zeros = jnp.zeros((LANES,), jnp.float32)
    neginf = jnp.full((LANES,), -jnp.inf, jnp.float32)
    seg_base = wid * SEG_PER_W

    pltpu.sync_copy(bounds_hbm, bv)
    pltpu.sync_copy(w_hbm, wv)

    def init_body(i, _):
        asum[pl.ds(i * LANES, LANES)] = zeros
        amax[pl.ds(i * LANES, LANES)] = neginf
        return 0

    lax.fori_loop(0, SEG_PER_W * HV, init_body, 0)

    def bound_body(j, _):
        bsm[j] = _extract_i32(bv, seg_base + j)
        return 0

    lax.fori_loop(0, SEG_PER_W + 1, bound_body, 0)

    start = bsm[0]
    end = bsm[SEG_PER_W]
    nchunks = (end - start + (CHUNK - 1)) // CHUNK

    def chunk_body(k, s_cur):
        lo_k = start + k * CHUNK
        hi_k = jnp.minimum(lo_k + CHUNK, end)
        base = jnp.minimum(lo_k, N_NODES - CHUNK)
        pltpu.sync_copy(x_hbm.at[pl.ds(base * HIDDEN, CHUNK * HIDDEN)], xbuf)

        def seg_cond(s):
            return jnp.logical_and(s < SEG_PER_W, bsm[s] < hi_k)

        def seg_body(s):
            r0 = jnp.maximum(bsm[s], lo_k)
            r1 = jnp.maximum(jnp.minimum(bsm[s + 1], hi_k), r0)
            soff = s * HIDDEN
            acc = tuple(
                [asum[pl.ds(soff + h * LANES, LANES)] for h in range(HV)]
                + [amax[pl.ds(soff + h * LANES, LANES)] for h in range(HV)]
            )

            def row_fn(r, a):
                xo = (r - base) * HIDDEN
                xs = [xbuf[pl.ds(xo + h * LANES, LANES)] for h in range(HV)]
                return tuple(
                    [a[h] + xs[h] for h in range(HV)]
                    + [jnp.maximum(a[HV + h], xs[h]) for h in range(HV)]
                )

            acc = lax.fori_loop(r0, r1, row_fn, acc)
            for h in range(HV):
                asum[pl.ds(soff + h * LANES, LANES)] = acc[h]
                amax[pl.ds(soff + h * LANES, LANES)] = acc[HV + h]
            return s + 1

        s_exit = lax.while_loop(seg_cond, seg_body, s_cur)
        return jnp.maximum(s_exit - 1, 0)

    lax.fori_loop(0, nchunks, chunk_body, 0)

    w0 = plsc.load_gather(wv, [_splat(1)])
    w1 = plsc.load_gather(wv, [_splat(2)])
    w2 = plsc.load_gather(wv, [_splat(3)])

    def out_body(s, _):
        cnt = (bsm[s + 1] - bsm[s]).astype(jnp.float32)
        cv = jnp.full((LANES,), cnt)
        has = cv > 0.0
        inv = 1.0 / jnp.maximum(cv, 1.0)
        for h in range(HV):
            off = s * HIDDEN + h * LANES
            sv = asum[pl.ds(off, LANES)]
            mx = jnp.where(has, amax[pl.ds(off, LANES)], zeros)
            obuf[pl.ds(off, LANES)] = (w0 * inv) * sv + w1 * mx + w2 * sv
        return 0

    lax.fori_loop(0, SEG_PER_W, out_body, 0)
    pltpu.sync_copy(obuf, out_hbm.at[pl.ds(seg_base * HIDDEN, SEG_PER_W * HIDDEN)])


def kernel(x, batch, mask, weights):
    del mask
    bounds = jnp.searchsorted(
        batch, jnp.arange(NUM_SEGMENTS + 1, dtype=jnp.int32), side="left"
    ).astype(jnp.int32)
    bounds = jnp.concatenate(
        [bounds, jnp.full((BPAD - NUM_SEGMENTS - 1,), N_NODES, jnp.int32)]
    )
    wpad = jnp.zeros((128,), jnp.float32).at[1:4].set(weights)
    out = _readout_kernel(x.reshape(-1), bounds, wpad)
    return out.reshape(NUM_SEGMENTS, HIDDEN)


# speedup vs baseline: 11.6184x; 2.7682x over previous
import functools

import jax
import jax.numpy as jnp
from jax import lax
from jax.experimental import pallas as pl
from jax.experimental.pallas import tpu as pltpu
from jax.experimental.pallas import tpu_sc as plsc

N_NODES = 100000
NUM_SEGMENTS = 1024
HIDDEN = 128
LANES = 16
HV = HIDDEN // LANES
SEG_PER_W = 32
CHUNK = 384
NWIN = 784
CPAD = 896
NQ = 48
SENT = 1 << 20


def _splat(val, dtype=jnp.int32):
    return jnp.full((LANES,), val, dtype)


def _extract_i32(ref, idx_scalar):
    g = plsc.load_gather(ref, [_splat(idx_scalar)])
    return jnp.max(g)


@functools.partial(
    pl.kernel,
    mesh=plsc.VectorSubcoreMesh(core_axis_name="c", subcore_axis_name="s"),
    out_type=jax.ShapeDtypeStruct((NUM_SEGMENTS * HIDDEN,), jnp.float32),
    compiler_params=pltpu.CompilerParams(needs_layout_passes=False),
    scratch_types=[
        pltpu.VMEM((CHUNK * HIDDEN,), jnp.float32),
        pltpu.VMEM((CPAD,), jnp.int32),
        pltpu.VMEM((NQ,), jnp.int32),
        pltpu.VMEM((NQ, 128), jnp.int32),
        pltpu.VMEM((128,), jnp.int32),
        pltpu.VMEM((128,), jnp.float32),
        pltpu.VMEM((SEG_PER_W * HIDDEN,), jnp.float32),
        pltpu.VMEM((SEG_PER_W * HIDDEN,), jnp.float32),
        pltpu.VMEM((SEG_PER_W * HIDDEN,), jnp.float32),
        pltpu.SMEM((40,), jnp.int32),
        pltpu.SemaphoreType.DMA,
    ],
)
def _readout_kernel(x_hbm, b2d_hbm, coarse_hbm, w_hbm, out_hbm,
                    xbuf, cv, widx, wins, jwbuf, wv, asum, amax, obuf, bsm,
                    sem):
    wid = lax.axis_index("c") * 16 + lax.axis_index("s")
    iota = jnp.arange(LANES, dtype=jnp.int32)
    zeros = jnp.zeros((LANES,), jnp.float32)
    neginf = jnp.full((LANES,), -jnp.inf, jnp.float32)
    seg_base = wid * SEG_PER_W

    pltpu.sync_copy(coarse_hbm, cv)
    pltpu.sync_copy(w_hbm, wv)

    def init_body(i, _):
        asum[pl.ds(i * LANES, LANES)] = zeros
        amax[pl.ds(i * LANES, LANES)] = neginf
        return 0

    lax.fori_loop(0, SEG_PER_W * HV, init_body, 0)

    for g in range(NQ // LANES):
        qv = seg_base + g * LANES + iota

        def bs_step(_, lh):
            lo, hi = lh
            mid = lax.div(lo + hi, 2)
            val = plsc.load_gather(cv, [mid])
            less = val < qv
            return (jnp.where(less, mid + 1, lo), jnp.where(less, hi, mid))

        lo0 = jnp.zeros((LANES,), jnp.int32)
        hi0 = jnp.full((LANES,), CPAD, jnp.int32)
        _, jq = lax.fori_loop(0, 10, bs_step, (lo0, hi0))
        jw = jnp.maximum(jq - 1, 0)
        widx[pl.ds(g * LANES, LANES)] = jw
        jwbuf[pl.ds(g * LANES, LANES)] = jw

    pltpu.async_copy(b2d_hbm.at[widx], wins, sem).wait()

    def refine_body(i, _):
        jw_i = _extract_i32(jwbuf, i)
        s_i = seg_base + i
        cacc = jnp.zeros((LANES,), jnp.int32)
        for h in range(HV):
            wvv = wins[i, pl.ds(h * LANES, LANES)]
            cacc = cacc + plsc.all_reduce_population_count(wvv < s_i)
        bsm[i] = jw_i * 128 + jnp.max(cacc)
        return 0

    lax.fori_loop(0, SEG_PER_W + 1, refine_body, 0)

    start = bsm[0]
    end = bsm[SEG_PER_W]
    nchunks = (end - start + (CHUNK - 1)) // CHUNK

    def chunk_body(k, s_cur):
        lo_k = start + k * CHUNK
        hi_k = jnp.minimum(lo_k + CHUNK, end)
        base = jnp.minimum(lo_k, N_NODES - CHUNK)
        pltpu.sync_copy(x_hbm.at[pl.ds(base * HIDDEN, CHUNK * HIDDEN)], xbuf)

        def seg_cond(s):
            return jnp.logical_and(s < SEG_PER_W, bsm[s] < hi_k)

        def seg_body(s):
            r0 = jnp.maximum(bsm[s], lo_k)
            r1 = jnp.maximum(jnp.minimum(bsm[s + 1], hi_k), r0)
            soff = s * HIDDEN
            acc = tuple(
                [asum[pl.ds(soff + h * LANES, LANES)] for h in range(HV)]
                + [amax[pl.ds(soff + h * LANES, LANES)] for h in range(HV)]
            )

            def row_fn(r, a):
                xo = (r - base) * HIDDEN
                xs = [xbuf[pl.ds(xo + h * LANES, LANES)] for h in range(HV)]
                return tuple(
                    [a[h] + xs[h] for h in range(HV)]
                    + [jnp.maximum(a[HV + h], xs[h]) for h in range(HV)]
                )

            acc = lax.fori_loop(r0, r1, row_fn, acc)
            for h in range(HV):
                asum[pl.ds(soff + h * LANES, LANES)] = acc[h]
                amax[pl.ds(soff + h * LANES, LANES)] = acc[HV + h]
            return s + 1

        s_exit = lax.while_loop(seg_cond, seg_body, s_cur)
        return jnp.maximum(s_exit - 1, 0)

    lax.fori_loop(0, nchunks, chunk_body, 0)

    w0 = plsc.load_gather(wv, [_splat(1)])
    w1 = plsc.load_gather(wv, [_splat(2)])
    w2 = plsc.load_gather(wv, [_splat(3)])

    def out_body(s, _):
        cnt = (bsm[s + 1] - bsm[s]).astype(jnp.float32)
        cvv = jnp.full((LANES,), cnt)
        has = cvv > 0.0
        inv = 1.0 / jnp.maximum(cvv, 1.0)
        for h in range(HV):
            off = s * HIDDEN + h * LANES
            sv = asum[pl.ds(off, LANES)]
            mx = jnp.where(has, amax[pl.ds(off, LANES)], zeros)
            obuf[pl.ds(off, LANES)] = (w0 * inv) * sv + w1 * mx + w2 * sv
        return 0

    lax.fori_loop(0, SEG_PER_W, out_body, 0)
    pltpu.sync_copy(obuf, out_hbm.at[pl.ds(seg_base * HIDDEN, SEG_PER_W * HIDDEN)])


def kernel(x, batch, mask, weights):
    del mask
    bpad = jnp.concatenate(
        [batch, jnp.full((NWIN * 128 - N_NODES,), SENT, jnp.int32)]
    )
    b2d = bpad.reshape(NWIN, 128)
    coarse = jnp.concatenate(
        [bpad[:: 128], jnp.full((CPAD - NWIN,), SENT, jnp.int32)]
    )
    wpad = jnp.zeros((128,), jnp.float32).at[1:4].set(weights)
    out = _readout_kernel(x.reshape(-1), b2d, coarse, wpad)
    return out.reshape(NUM_SEGMENTS, HIDDEN)


# speedup vs baseline: 14.9011x; 1.2826x over previous
import functools

import jax
import jax.numpy as jnp
from jax import lax
from jax.experimental import pallas as pl
from jax.experimental.pallas import tpu as pltpu
from jax.experimental.pallas import tpu_sc as plsc

N_NODES = 100000
NUM_SEGMENTS = 1024
HIDDEN = 128
LANES = 16
HV = HIDDEN // LANES
SEG_PER_W = 32
CHUNK = 256
NWIN = 784
CPAD = 896
NQ = 48
SENT = 1 << 20


def _splat(val, dtype=jnp.int32):
    return jnp.full((LANES,), val, dtype)


def _extract_i32(ref, idx_scalar):
    g = plsc.load_gather(ref, [_splat(idx_scalar)])
    return jnp.max(g)


@functools.partial(
    pl.kernel,
    mesh=plsc.VectorSubcoreMesh(core_axis_name="c", subcore_axis_name="s"),
    out_type=jax.ShapeDtypeStruct((NUM_SEGMENTS * HIDDEN,), jnp.float32),
    compiler_params=pltpu.CompilerParams(needs_layout_passes=False),
    scratch_types=[
        pltpu.VMEM((CHUNK * HIDDEN,), jnp.float32),
        pltpu.VMEM((CHUNK * HIDDEN,), jnp.float32),
        pltpu.VMEM((CPAD,), jnp.int32),
        pltpu.VMEM((NQ,), jnp.int32),
        pltpu.VMEM((NQ, 128), jnp.int32),
        pltpu.VMEM((128,), jnp.int32),
        pltpu.VMEM((128,), jnp.float32),
        pltpu.VMEM((SEG_PER_W * HIDDEN,), jnp.float32),
        pltpu.VMEM((SEG_PER_W * HIDDEN,), jnp.float32),
        pltpu.VMEM((SEG_PER_W * HIDDEN,), jnp.float32),
        pltpu.SMEM((40,), jnp.int32),
        pltpu.SemaphoreType.DMA,
        pltpu.SemaphoreType.DMA,
        pltpu.SemaphoreType.DMA,
    ],
)
def _readout_kernel(x_hbm, b2d_hbm, coarse_hbm, w_hbm, out_hbm,
                    xbuf0, xbuf1, cv, widx, wins, jwbuf, wv, asum, amax,
                    obuf, bsm, sem, sem0, sem1):
    wid = lax.axis_index("c") * 16 + lax.axis_index("s")
    iota = jnp.arange(LANES, dtype=jnp.int32)
    zeros = jnp.zeros((LANES,), jnp.float32)
    neginf = jnp.full((LANES,), -jnp.inf, jnp.float32)
    seg_base = wid * SEG_PER_W

    pltpu.sync_copy(coarse_hbm, cv)
    pltpu.sync_copy(w_hbm, wv)

    def init_body(i, _):
        asum[pl.ds(i * LANES, LANES)] = zeros
        amax[pl.ds(i * LANES, LANES)] = neginf
        return 0

    lax.fori_loop(0, SEG_PER_W * HV, init_body, 0)

    for g in range(NQ // LANES):
        qv = seg_base + g * LANES + iota

        def bs_step(_, lh):
            lo, hi = lh
            mid = lax.div(lo + hi, 2)
            val = plsc.load_gather(cv, [mid])
            less = val < qv
            return (jnp.where(less, mid + 1, lo), jnp.where(less, hi, mid))

        lo0 = jnp.zeros((LANES,), jnp.int32)
        hi0 = jnp.full((LANES,), CPAD, jnp.int32)
        _, jq = lax.fori_loop(0, 10, bs_step, (lo0, hi0))
        jw = jnp.maximum(jq - 1, 0)
        widx[pl.ds(g * LANES, LANES)] = jw
        jwbuf[pl.ds(g * LANES, LANES)] = jw

    pltpu.async_copy(b2d_hbm.at[widx], wins, sem).wait()

    def refine_body(i, _):
        jw_i = _extract_i32(jwbuf, i)
        s_i = seg_base + i
        cacc = jnp.zeros((LANES,), jnp.int32)
        for h in range(HV):
            wvv = wins[i, pl.ds(h * LANES, LANES)]
            cacc = cacc + plsc.all_reduce_population_count(wvv < s_i)
        bsm[i] = jw_i * 128 + jnp.max(cacc)
        return 0

    lax.fori_loop(0, SEG_PER_W + 1, refine_body, 0)

    start = bsm[0]
    end = bsm[SEG_PER_W]
    nchunks = (end - start + (CHUNK - 1)) // CHUNK
    nch2 = 2 * ((nchunks + 1) // 2)

    def dma_start(k, buf, dsem):
        base = jnp.minimum(start + k * CHUNK, N_NODES - CHUNK)
        pltpu.async_copy(
            x_hbm.at[pl.ds(base * HIDDEN, CHUNK * HIDDEN)], buf, dsem)

    def dma_wait(buf, dsem):
        pltpu.make_async_copy(
            x_hbm.at[pl.ds(0, CHUNK * HIDDEN)], buf, dsem).wait()

    @pl.when(nchunks > 0)
    def _():
        dma_start(0, xbuf0, sem0)
        dma_start(1, xbuf1, sem1)

    def process(k, buf, dsem, s_cur):
        lo_k = start + k * CHUNK
        hi_k = jnp.minimum(lo_k + CHUNK, end)
        base = jnp.minimum(lo_k, N_NODES - CHUNK)
        dma_wait(buf, dsem)

        def seg_cond(s):
            return jnp.logical_and(s < SEG_PER_W, bsm[s] < hi_k)

        def seg_body(s):
            r0 = jnp.maximum(bsm[s], lo_k)
            r1 = jnp.maximum(jnp.minimum(bsm[s + 1], hi_k), r0)
            soff = s * HIDDEN
            acc = tuple(
                [asum[pl.ds(soff + h * LANES, LANES)] for h in range(HV)]
                + [amax[pl.ds(soff + h * LANES, LANES)] for h in range(HV)]
            )

            def row_fn(r, a):
                xo = (r - base) * HIDDEN
                xs = [buf[pl.ds(xo + h * LANES, LANES)] for h in range(HV)]
                return tuple(
                    [a[h] + xs[h] for h in range(HV)]
                    + [jnp.maximum(a[HV + h], xs[h]) for h in range(HV)]
                )

            acc = lax.fori_loop(r0, r1, row_fn, acc)
            for h in range(HV):
                asum[pl.ds(soff + h * LANES, LANES)] = acc[h]
                amax[pl.ds(soff + h * LANES, LANES)] = acc[HV + h]
            return s + 1

        s_exit = lax.while_loop(seg_cond, seg_body, s_cur)

        @pl.when(k + 2 < nch2)
        def _():
            dma_start(k + 2, buf, dsem)

        return jnp.maximum(s_exit - 1, 0)

    def pair_body(k2, s_cur):
        k0 = 2 * k2
        s_cur = process(k0, xbuf0, sem0, s_cur)
        s_cur = process(k0 + 1, xbuf1, sem1, s_cur)
        return s_cur

    lax.fori_loop(0, nch2 // 2, pair_body, 0)

    w0 = plsc.load_gather(wv, [_splat(1)])
    w1 = plsc.load_gather(wv, [_splat(2)])
    w2 = plsc.load_gather(wv, [_splat(3)])

    def out_body(s, _):
        cnt = (bsm[s + 1] - bsm[s]).astype(jnp.float32)
        cvv = jnp.full((LANES,), cnt)
        has = cvv > 0.0
        inv = 1.0 / jnp.maximum(cvv, 1.0)
        for h in range(HV):
            off = s * HIDDEN + h * LANES
            sv = asum[pl.ds(off, LANES)]
            mx = jnp.where(has, amax[pl.ds(off, LANES)], zeros)
            obuf[pl.ds(off, LANES)] = (w0 * inv) * sv + w1 * mx + w2 * sv
        return 0

    lax.fori_loop(0, SEG_PER_W, out_body, 0)
    pltpu.sync_copy(obuf, out_hbm.at[pl.ds(seg_base * HIDDEN, SEG_PER_W * HIDDEN)])


def kernel(x, batch, mask, weights):
    del mask
    bpad = jnp.concatenate(
        [batch, jnp.full((NWIN * 128 - N_NODES,), SENT, jnp.int32)]
    )
    b2d = bpad.reshape(NWIN, 128)
    coarse = jnp.concatenate(
        [bpad[:: 128], jnp.full((CPAD - NWIN,), SENT, jnp.int32)]
    )
    wpad = jnp.zeros((128,), jnp.float32).at[1:4].set(weights)
    out = _readout_kernel(x.reshape(-1), b2d, coarse, wpad)
    return out.reshape(NUM_SEGMENTS, HIDDEN)
